# Initial kernel scaffold; baseline (speedup 1.0000x reference)
import functools

import jax
import jax.numpy as jnp
from jax import lax
from jax.experimental import pallas as pl
from jax.experimental.pallas import tpu as pltpu
from jax.experimental.pallas import tpu_sc as plsc

DIM = 32
SEQ = 50
BATCH = 4096
N = BATCH * SEQ

_info = plsc.get_sparse_core_info()
NC, NS = _info.num_cores, _info.num_subcores
NW = NC * NS
BW = N // NW
NCH = 2
CH = BW // NCH


def _sc_gather(idx_flat, emb):
    mesh = plsc.VectorSubcoreMesh(core_axis_name="c", subcore_axis_name="s")

    @functools.partial(
        pl.kernel,
        mesh=mesh,
        out_type=jax.ShapeDtypeStruct((N, DIM), jnp.float32),
        scratch_types=[
            pltpu.VMEM((BW,), jnp.int32),
            pltpu.VMEM((CH, DIM), jnp.float32),
            pltpu.SemaphoreType.DMA,
        ],
    )
    def k(idx_hbm, table_hbm, out_hbm, idx_v, rows_v, sem):
        wid = lax.axis_index("s") * NC + lax.axis_index("c")
        base = wid * BW
        pltpu.sync_copy(idx_hbm.at[pl.ds(base, BW)], idx_v)
        for c in range(NCH):
            pltpu.async_copy(
                table_hbm.at[idx_v.at[pl.ds(c * CH, CH)]], rows_v, sem
            ).wait()
            pltpu.sync_copy(rows_v, out_hbm.at[pl.ds(base + c * CH, CH)])

    return k(idx_flat, emb)


_B_BLK = 512


def _mlp_body(x_ref, w1_ref, b1_ref, w2_ref, b2_ref, out_ref):
    h = jnp.dot(x_ref[...], w1_ref[...],
                preferred_element_type=jnp.float32,
                precision=lax.Precision.HIGHEST)
    h = jnp.maximum(h + b1_ref[...], 0.0)
    o = jnp.dot(h, w2_ref[...],
                preferred_element_type=jnp.float32,
                precision=lax.Precision.HIGHEST)
    out_ref[...] = jax.nn.sigmoid(o + b2_ref[...])


def _tc_mlp(x, W1, b1, W2, b2):
    grid = (BATCH // _B_BLK,)
    return pl.pallas_call(
        _mlp_body,
        grid=grid,
        in_specs=[
            pl.BlockSpec((_B_BLK, SEQ * DIM), lambda i: (i, 0)),
            pl.BlockSpec((SEQ * DIM, 10), lambda i: (0, 0)),
            pl.BlockSpec((1, 10), lambda i: (0, 0)),
            pl.BlockSpec((10, 5), lambda i: (0, 0)),
            pl.BlockSpec((1, 5), lambda i: (0, 0)),
        ],
        out_specs=pl.BlockSpec((_B_BLK, 5), lambda i: (i, 0)),
        out_shape=jax.ShapeDtypeStruct((BATCH, 5), jnp.float32),
    )(x, W1, b1, W2, b2)


def kernel(indices, emb, W1, b1, W2, b2):
    idx_flat = indices.astype(jnp.int32).reshape(N)
    rows = _sc_gather(idx_flat, emb)
    x = rows.reshape(BATCH, SEQ * DIM)
    return _tc_mlp(x, W1, b1.reshape(1, 10), W2, b2.reshape(1, 5))


# speedup vs baseline: 8.8573x; 8.8573x over previous
import functools

import jax
import jax.numpy as jnp
from jax import lax
from jax.experimental import pallas as pl
from jax.experimental.pallas import tpu as pltpu
from jax.experimental.pallas import tpu_sc as plsc

DIM = 32
SEQ = 50
BATCH = 4096
N = BATCH * SEQ

_info = plsc.get_sparse_core_info()
NC, NS = _info.num_cores, _info.num_subcores
NW = NC * NS
BW = N // NW
NCH = 2
CH = BW // NCH


def _sc_gather(idx_flat, emb):
    mesh = plsc.VectorSubcoreMesh(core_axis_name="c", subcore_axis_name="s")

    @functools.partial(
        pl.kernel,
        mesh=mesh,
        out_type=jax.ShapeDtypeStruct((N, DIM), jnp.float32),
        compiler_params=pltpu.CompilerParams(use_tc_tiling_on_sc=False),
        scratch_types=[
            pltpu.VMEM((BW,), jnp.int32),
            pltpu.VMEM((CH, DIM), jnp.float32),
            pltpu.SemaphoreType.DMA,
        ],
    )
    def k(idx_hbm, table_hbm, out_hbm, idx_v, rows_v, sem):
        wid = lax.axis_index("s") * NC + lax.axis_index("c")
        base = wid * BW
        pltpu.sync_copy(idx_hbm.at[pl.ds(base, BW)], idx_v)
        for c in range(NCH):
            pltpu.async_copy(
                table_hbm.at[idx_v.at[pl.ds(c * CH, CH)]], rows_v, sem
            ).wait()
            pltpu.sync_copy(rows_v, out_hbm.at[pl.ds(base + c * CH, CH)])

    return k(idx_flat, emb)


_B_BLK = 512


def _mlp_body(x_ref, w1_ref, b1_ref, w2_ref, b2_ref, out_ref):
    h = jnp.dot(x_ref[...], w1_ref[...],
                preferred_element_type=jnp.float32,
                precision=lax.Precision.HIGHEST)
    h = jnp.maximum(h + b1_ref[...], 0.0)
    o = jnp.dot(h, w2_ref[...],
                preferred_element_type=jnp.float32,
                precision=lax.Precision.HIGHEST)
    out_ref[...] = jax.nn.sigmoid(o + b2_ref[...])


def _tc_mlp(x, W1, b1, W2, b2):
    grid = (BATCH // _B_BLK,)
    return pl.pallas_call(
        _mlp_body,
        grid=grid,
        in_specs=[
            pl.BlockSpec((_B_BLK, SEQ * DIM), lambda i: (i, 0)),
            pl.BlockSpec((SEQ * DIM, 10), lambda i: (0, 0)),
            pl.BlockSpec((1, 10), lambda i: (0, 0)),
            pl.BlockSpec((10, 5), lambda i: (0, 0)),
            pl.BlockSpec((1, 5), lambda i: (0, 0)),
        ],
        out_specs=pl.BlockSpec((_B_BLK, 5), lambda i: (i, 0)),
        out_shape=jax.ShapeDtypeStruct((BATCH, 5), jnp.float32),
    )(x, W1, b1, W2, b2)


def kernel(indices, emb, W1, b1, W2, b2):
    idx_flat = indices.astype(jnp.int32).reshape(N)
    rows = _sc_gather(idx_flat, emb)
    x = rows.reshape(BATCH, SEQ * DIM)
    return _tc_mlp(x, W1, b1.reshape(1, 10), W2, b2.reshape(1, 5))
